# baseline (speedup 1.0000x reference)
import functools

import jax
import jax.numpy as jnp
from jax import lax
from jax.experimental import pallas as pl
from jax.experimental.pallas import tpu as pltpu
from jax.experimental.pallas import tpu_sc as plsc

_BATCH = 16384
_DIM = 64
_PAD_DIM = 128
_TABLE_ROWS = 100000
_GAMMA = 12.0

_NC = 2
_NS = 16
_L = 16
_NW = _NC * _NS
_ROWS_PER_W = _BATCH // _NW
_CHUNK = 128
_NCHUNK = _ROWS_PER_W // _CHUNK


_RB = 5000
_RG = _TABLE_ROWS // _RB


def _in_dma(src, g, buf, sem):
    return pltpu.make_async_copy(src.at[pl.ds(g * _RB, _RB)], buf, sem)


def _out_dma(dst, g, buf, sem):
    return pltpu.make_async_copy(buf, dst.at[pl.ds(g * _RB, _RB)], sem)


def _repack_body(ent_ref, rel_ref, ent_o, rel_o,
                 s1a, s1b, s2a, s2b, sin_a, sin_b, sout_a, sout_b):
    for src, dst in ((ent_ref, ent_o), (rel_ref, rel_o)):
        _in_dma(src, 0, s1a, sin_a).start()

        def body(k, carry):
            g0 = 2 * k
            g1 = g0 + 1
            _in_dma(src, g1, s1b, sin_b).start()
            _in_dma(src, g0, s1a, sin_a).wait()

            @pl.when(k > 0)
            def _():
                _out_dma(dst, g0 - 2, s2a, sout_a).wait()

            s2a[:, :_DIM] = s1a[...]
            _out_dma(dst, g0, s2a, sout_a).start()

            @pl.when(k < _RG // 2 - 1)
            def _():
                _in_dma(src, g0 + 2, s1a, sin_a).start()

            _in_dma(src, g1, s1b, sin_b).wait()

            @pl.when(k > 0)
            def _():
                _out_dma(dst, g1 - 2, s2b, sout_b).wait()

            s2b[:, :_DIM] = s1b[...]
            _out_dma(dst, g1, s2b, sout_b).start()
            return carry

        lax.fori_loop(0, _RG // 2, body, 0)
        _out_dma(dst, _RG - 2, s2a, sout_a).wait()
        _out_dma(dst, _RG - 1, s2b, sout_b).wait()


_repack = pl.pallas_call(
    _repack_body,
    in_specs=[
        pl.BlockSpec(memory_space=pltpu.HBM),
        pl.BlockSpec(memory_space=pltpu.HBM),
    ],
    out_specs=[
        pl.BlockSpec(memory_space=pltpu.HBM),
        pl.BlockSpec(memory_space=pltpu.HBM),
    ],
    out_shape=(
        jax.ShapeDtypeStruct((_TABLE_ROWS, _PAD_DIM), jnp.float32),
        jax.ShapeDtypeStruct((_TABLE_ROWS, _PAD_DIM), jnp.float32),
    ),
    scratch_shapes=[
        pltpu.VMEM((_RB, _DIM), jnp.float32),
        pltpu.VMEM((_RB, _DIM), jnp.float32),
        pltpu.VMEM((_RB, _PAD_DIM), jnp.float32),
        pltpu.VMEM((_RB, _PAD_DIM), jnp.float32),
        pltpu.SemaphoreType.DMA,
        pltpu.SemaphoreType.DMA,
        pltpu.SemaphoreType.DMA,
        pltpu.SemaphoreType.DMA,
    ],
)


def _compute_chunk(rows_h, rows_r, rows_t, out_v, out_base):
    lane = lax.iota(jnp.int32, _L)

    def block_body(b, carry):
        acc = jnp.zeros((_L,), jnp.float32)
        for l in range(_L):
            row = b * _L + l
            psum = jnp.zeros((_L,), jnp.float32)
            for j in range(_DIM // _L):
                sl = pl.ds(j * _L, _L)
                hv = rows_h[row, sl]
                rv = rows_r[row, sl]
                tv = rows_t[row, sl]
                psum = psum + jnp.abs(hv + rv - tv)
            total = jnp.sum(psum) - _GAMMA
            acc = jnp.where(lane == l, total, acc)
        out_v[pl.ds(out_base + b * _L, _L)] = acc
        return carry

    lax.fori_loop(0, _CHUNK // _L, block_body, 0)


def _body(hidx_hbm, ridx_hbm, tidx_hbm, ent_hbm, rel_hbm, out_hbm,
          idx_h, idx_r, idx_t, rows_h, rows_r, rows_t, out_v, sem):
    wid = lax.axis_index("s") * _NC + lax.axis_index("c")
    base = wid * _ROWS_PER_W

    for c in range(_NCHUNK):
        src = pl.ds(base + c * _CHUNK, _CHUNK)
        pltpu.sync_copy(hidx_hbm.at[src], idx_h.at[c])
        pltpu.sync_copy(ridx_hbm.at[src], idx_r.at[c])
        pltpu.sync_copy(tidx_hbm.at[src], idx_t.at[c])

    for c in range(_NCHUNK):
        cp_h = pltpu.async_copy(ent_hbm.at[idx_h.at[c]], rows_h, sem)
        cp_r = pltpu.async_copy(rel_hbm.at[idx_r.at[c]], rows_r, sem)
        cp_t = pltpu.async_copy(ent_hbm.at[idx_t.at[c]], rows_t, sem)
        cp_h.wait()
        cp_r.wait()
        cp_t.wait()
        _compute_chunk(rows_h, rows_r, rows_t, out_v, c * _CHUNK)

    pltpu.sync_copy(out_v, out_hbm.at[pl.ds(base, _ROWS_PER_W)])


@functools.partial(
    pl.kernel,
    out_type=jax.ShapeDtypeStruct((_BATCH,), jnp.float32),
    scratch_types=[
        pltpu.VMEM((_NCHUNK, _CHUNK), jnp.int32),
        pltpu.VMEM((_NCHUNK, _CHUNK), jnp.int32),
        pltpu.VMEM((_NCHUNK, _CHUNK), jnp.int32),
        pltpu.VMEM((_CHUNK, _PAD_DIM), jnp.float32),
        pltpu.VMEM((_CHUNK, _PAD_DIM), jnp.float32),
        pltpu.VMEM((_CHUNK, _PAD_DIM), jnp.float32),
        pltpu.VMEM((_ROWS_PER_W,), jnp.float32),
        pltpu.SemaphoreType.DMA,
    ],
    mesh=plsc.VectorSubcoreMesh(core_axis_name="c", subcore_axis_name="s"),
    compiler_params=pltpu.CompilerParams(
        needs_layout_passes=False, use_tc_tiling_on_sc=False
    ),
)
def _transe_sc(*args):
    _body(*args)


def kernel(pos_sample, ent_embd, rel_embd):
    ent_p, rel_p = _repack(ent_embd, rel_embd)
    h_idx = pos_sample[:, 0]
    r_idx = pos_sample[:, 1]
    t_idx = pos_sample[:, 2]
    score = _transe_sc(h_idx, r_idx, t_idx, ent_p, rel_p)
    return score[:, None]


# speedup vs baseline: 1.8990x; 1.8990x over previous
import functools

import jax
import jax.numpy as jnp
from jax import lax
from jax.experimental import pallas as pl
from jax.experimental.pallas import tpu as pltpu
from jax.experimental.pallas import tpu_sc as plsc

_BATCH = 16384
_DIM = 64
_PAD_DIM = 128
_TABLE_ROWS = 100000
_GAMMA = 12.0

_NC = 2
_NS = 16
_L = 16
_NW = _NC * _NS
_ROWS_PER_W = _BATCH // _NW
_CHUNK = 128
_NCHUNK = _ROWS_PER_W // _CHUNK


_TB = 8192
_TG = (_TABLE_ROWS + _TB - 1) // _TB


def _transpose_body(ent_ref, rel_ref, ent_o, rel_o):
    ent_o[:, :_DIM] = jnp.swapaxes(ent_ref[...], 0, 1)
    rel_o[:, :_DIM] = jnp.swapaxes(rel_ref[...], 0, 1)


_transpose_tables = pl.pallas_call(
    _transpose_body,
    grid=(_TG,),
    in_specs=[
        pl.BlockSpec((_DIM, _TB), lambda i: (0, i)),
        pl.BlockSpec((_DIM, _TB), lambda i: (0, i)),
    ],
    out_specs=[
        pl.BlockSpec((_TB, _PAD_DIM), lambda i: (i, 0)),
        pl.BlockSpec((_TB, _PAD_DIM), lambda i: (i, 0)),
    ],
    out_shape=(
        jax.ShapeDtypeStruct((_TABLE_ROWS, _PAD_DIM), jnp.float32),
        jax.ShapeDtypeStruct((_TABLE_ROWS, _PAD_DIM), jnp.float32),
    ),
)


def _compute_chunk(rows_h, rows_r, rows_t, out_v, out_base):
    lane = lax.iota(jnp.int32, _L)

    def block_body(b, carry):
        acc = jnp.zeros((_L,), jnp.float32)
        for l in range(_L):
            row = b * _L + l
            psum = jnp.zeros((_L,), jnp.float32)
            for j in range(_DIM // _L):
                sl = pl.ds(j * _L, _L)
                hv = rows_h[row, sl]
                rv = rows_r[row, sl]
                tv = rows_t[row, sl]
                psum = psum + jnp.abs(hv + rv - tv)
            total = jnp.sum(psum) - _GAMMA
            acc = jnp.where(lane == l, total, acc)
        out_v[pl.ds(out_base + b * _L, _L)] = acc
        return carry

    lax.fori_loop(0, _CHUNK // _L, block_body, 0)


def _body(hidx_hbm, ridx_hbm, tidx_hbm, ent_hbm, rel_hbm, out_hbm,
          idx_h, idx_r, idx_t, rows_h, rows_r, rows_t, out_v, sem):
    wid = lax.axis_index("s") * _NC + lax.axis_index("c")
    base = wid * _ROWS_PER_W

    for c in range(_NCHUNK):
        src = pl.ds(base + c * _CHUNK, _CHUNK)
        pltpu.sync_copy(hidx_hbm.at[src], idx_h.at[c])
        pltpu.sync_copy(ridx_hbm.at[src], idx_r.at[c])
        pltpu.sync_copy(tidx_hbm.at[src], idx_t.at[c])

    for c in range(_NCHUNK):
        cp_h = pltpu.async_copy(ent_hbm.at[idx_h.at[c]], rows_h, sem)
        cp_r = pltpu.async_copy(rel_hbm.at[idx_r.at[c]], rows_r, sem)
        cp_t = pltpu.async_copy(ent_hbm.at[idx_t.at[c]], rows_t, sem)
        cp_h.wait()
        cp_r.wait()
        cp_t.wait()
        _compute_chunk(rows_h, rows_r, rows_t, out_v, c * _CHUNK)

    pltpu.sync_copy(out_v, out_hbm.at[pl.ds(base, _ROWS_PER_W)])


@functools.partial(
    pl.kernel,
    out_type=jax.ShapeDtypeStruct((_BATCH,), jnp.float32),
    scratch_types=[
        pltpu.VMEM((_NCHUNK, _CHUNK), jnp.int32),
        pltpu.VMEM((_NCHUNK, _CHUNK), jnp.int32),
        pltpu.VMEM((_NCHUNK, _CHUNK), jnp.int32),
        pltpu.VMEM((_CHUNK, _PAD_DIM), jnp.float32),
        pltpu.VMEM((_CHUNK, _PAD_DIM), jnp.float32),
        pltpu.VMEM((_CHUNK, _PAD_DIM), jnp.float32),
        pltpu.VMEM((_ROWS_PER_W,), jnp.float32),
        pltpu.SemaphoreType.DMA,
    ],
    mesh=plsc.VectorSubcoreMesh(core_axis_name="c", subcore_axis_name="s"),
    compiler_params=pltpu.CompilerParams(
        needs_layout_passes=False, use_tc_tiling_on_sc=False
    ),
)
def _transe_sc(*args):
    _body(*args)


def kernel(pos_sample, ent_embd, rel_embd):
    ent_p, rel_p = _transpose_tables(ent_embd.T, rel_embd.T)
    h_idx = pos_sample[:, 0]
    r_idx = pos_sample[:, 1]
    t_idx = pos_sample[:, 2]
    score = _transe_sc(h_idx, r_idx, t_idx, ent_p, rel_p)
    return score[:, None]


# speedup vs baseline: 2.0099x; 1.0584x over previous
import functools

import jax
import jax.numpy as jnp
from jax import lax
from jax.experimental import pallas as pl
from jax.experimental.pallas import tpu as pltpu
from jax.experimental.pallas import tpu_sc as plsc

_BATCH = 16384
_DIM = 64
_PAD_DIM = 128
_TABLE_ROWS = 100000
_GAMMA = 12.0

_NC = 2
_NS = 16
_L = 16
_NW = _NC * _NS
_ROWS_PER_W = _BATCH // _NW
_CHUNK = 128
_NCHUNK = _ROWS_PER_W // _CHUNK

_TB = 8192
_TG = (_TABLE_ROWS + _TB - 1) // _TB


def _transpose_body(ent_ref, rel_ref, ent_o, rel_o):
    ent_o[:, :_DIM] = jnp.swapaxes(ent_ref[...], 0, 1)
    rel_o[:, :_DIM] = jnp.swapaxes(rel_ref[...], 0, 1)


_transpose_tables = pl.pallas_call(
    _transpose_body,
    grid=(_TG,),
    in_specs=[
        pl.BlockSpec((_DIM, _TB), lambda i: (0, i)),
        pl.BlockSpec((_DIM, _TB), lambda i: (0, i)),
    ],
    out_specs=[
        pl.BlockSpec((_TB, _PAD_DIM), lambda i: (i, 0)),
        pl.BlockSpec((_TB, _PAD_DIM), lambda i: (i, 0)),
    ],
    out_shape=(
        jax.ShapeDtypeStruct((_TABLE_ROWS, _PAD_DIM), jnp.float32),
        jax.ShapeDtypeStruct((_TABLE_ROWS, _PAD_DIM), jnp.float32),
    ),
)


def _compute_chunk(rows_h, rows_r, rows_t, out_v, out_base):
    lane = lax.iota(jnp.int32, _L)

    def block_body(b, carry):
        acc = jnp.zeros((_L,), jnp.float32)
        for l in range(_L):
            row = b * _L + l
            psum = jnp.zeros((_L,), jnp.float32)
            for j in range(_DIM // _L):
                sl = pl.ds(j * _L, _L)
                hv = rows_h[row, sl]
                rv = rows_r[row, sl]
                tv = rows_t[row, sl]
                psum = psum + jnp.abs(hv + rv - tv)
            total = jnp.sum(psum) - _GAMMA
            acc = jnp.where(lane == l, total, acc)
        out_v[pl.ds(out_base + b * _L, _L)] = acc
        return carry

    lax.fori_loop(0, _CHUNK // _L, block_body, 0)


def _body(hidx_hbm, ridx_hbm, tidx_hbm, ent_hbm, rel_hbm, out_hbm,
          idx_h, idx_r, idx_t,
          rows_ha, rows_ra, rows_ta, rows_hb, rows_rb, rows_tb,
          out_v, sem_a, sem_b):
    wid = lax.axis_index("s") * _NC + lax.axis_index("c")
    base = wid * _ROWS_PER_W

    for c in range(_NCHUNK):
        src = pl.ds(base + c * _CHUNK, _CHUNK)
        pltpu.sync_copy(hidx_hbm.at[src], idx_h.at[c])
        pltpu.sync_copy(ridx_hbm.at[src], idx_r.at[c])
        pltpu.sync_copy(tidx_hbm.at[src], idx_t.at[c])

    bufs = ((rows_ha, rows_ra, rows_ta), (rows_hb, rows_rb, rows_tb))
    sems = (sem_a, sem_b)

    def fire(c, bset, sem):
        return (
            pltpu.async_copy(ent_hbm.at[idx_h.at[c]], bset[0], sem),
            pltpu.async_copy(rel_hbm.at[idx_r.at[c]], bset[1], sem),
            pltpu.async_copy(ent_hbm.at[idx_t.at[c]], bset[2], sem),
        )

    cps = fire(0, bufs[0], sems[0])
    for c in range(_NCHUNK):
        nxt = None
        if c + 1 < _NCHUNK:
            nxt = fire(c + 1, bufs[(c + 1) % 2], sems[(c + 1) % 2])
        for cp in cps:
            cp.wait()
        _compute_chunk(*bufs[c % 2], out_v, c * _CHUNK)
        cps = nxt

    pltpu.sync_copy(out_v, out_hbm.at[pl.ds(base, _ROWS_PER_W)])


@functools.partial(
    pl.kernel,
    out_type=jax.ShapeDtypeStruct((_BATCH,), jnp.float32),
    scratch_types=[
        pltpu.VMEM((_NCHUNK, _CHUNK), jnp.int32),
        pltpu.VMEM((_NCHUNK, _CHUNK), jnp.int32),
        pltpu.VMEM((_NCHUNK, _CHUNK), jnp.int32),
        pltpu.VMEM((_CHUNK, _PAD_DIM), jnp.float32),
        pltpu.VMEM((_CHUNK, _PAD_DIM), jnp.float32),
        pltpu.VMEM((_CHUNK, _PAD_DIM), jnp.float32),
        pltpu.VMEM((_CHUNK, _PAD_DIM), jnp.float32),
        pltpu.VMEM((_CHUNK, _PAD_DIM), jnp.float32),
        pltpu.VMEM((_CHUNK, _PAD_DIM), jnp.float32),
        pltpu.VMEM((_ROWS_PER_W,), jnp.float32),
        pltpu.SemaphoreType.DMA,
        pltpu.SemaphoreType.DMA,
    ],
    mesh=plsc.VectorSubcoreMesh(core_axis_name="c", subcore_axis_name="s"),
    compiler_params=pltpu.CompilerParams(
        needs_layout_passes=False, use_tc_tiling_on_sc=False
    ),
)
def _transe_sc(*args):
    _body(*args)


def kernel(pos_sample, ent_embd, rel_embd):
    ent_p, rel_p = _transpose_tables(ent_embd.T, rel_embd.T)
    h_idx = pos_sample[:, 0]
    r_idx = pos_sample[:, 1]
    t_idx = pos_sample[:, 2]
    score = _transe_sc(h_idx, r_idx, t_idx, ent_p, rel_p)
    return score[:, None]


# speedup vs baseline: 4.5430x; 2.2603x over previous
import functools

import jax
import jax.numpy as jnp
from jax import lax
from jax.experimental import pallas as pl
from jax.experimental.pallas import tpu as pltpu
from jax.experimental.pallas import tpu_sc as plsc

_BATCH = 16384
_DIM = 64
_PAD_DIM = 128
_TABLE_ROWS = 100000
_GAMMA = 12.0

_NC = 2
_NS = 16
_L = 16
_NW = _NC * _NS
_ROWS_PER_W = _BATCH // _NW
_CHUNK = 128
_NCHUNK = _ROWS_PER_W // _CHUNK

_TB = 8192
_TG = 7
_HALF = _TG * _TB


def _transpose_body(e1_ref, e2_ref, r1_ref, r2_ref, ent_o, rel_o):
    ent_o[:, :_DIM] = jnp.swapaxes(e1_ref[...], 0, 1)
    ent_o[:, _DIM:] = jnp.swapaxes(e2_ref[...], 0, 1)
    rel_o[:, :_DIM] = jnp.swapaxes(r1_ref[...], 0, 1)
    rel_o[:, _DIM:] = jnp.swapaxes(r2_ref[...], 0, 1)


_transpose_tables = pl.pallas_call(
    _transpose_body,
    grid=(_TG,),
    in_specs=[
        pl.BlockSpec((_DIM, _TB), lambda i: (0, i)),
        pl.BlockSpec(
            (_DIM, _TB),
            lambda i: (0, jnp.minimum(i + _TG, (_TABLE_ROWS - 1) // _TB)),
        ),
        pl.BlockSpec((_DIM, _TB), lambda i: (0, i)),
        pl.BlockSpec(
            (_DIM, _TB),
            lambda i: (0, jnp.minimum(i + _TG, (_TABLE_ROWS - 1) // _TB)),
        ),
    ],
    out_specs=[
        pl.BlockSpec((_TB, _PAD_DIM), lambda i: (i, 0)),
        pl.BlockSpec((_TB, _PAD_DIM), lambda i: (i, 0)),
    ],
    out_shape=(
        jax.ShapeDtypeStruct((_HALF, _PAD_DIM), jnp.float32),
        jax.ShapeDtypeStruct((_HALF, _PAD_DIM), jnp.float32),
    ),
)


def _compute_chunk(c, rows_h, rows_r, rows_t, idx_h, idx_r, idx_t,
                   out_v, out_base):
    lane = lax.iota(jnp.int32, _L)

    def block_body(b, carry):
        bsl = pl.ds(b * _L, _L)
        par_h = jnp.where(idx_h[c, bsl] >= _HALF, _DIM, 0)
        par_r = jnp.where(idx_r[c, bsl] >= _HALF, _DIM, 0)
        par_t = jnp.where(idx_t[c, bsl] >= _HALF, _DIM, 0)
        acc = jnp.zeros((_L,), jnp.float32)
        for l in range(_L):
            row = b * _L + l
            ch = par_h[l]
            cr = par_r[l]
            ct = par_t[l]
            psum = jnp.zeros((_L,), jnp.float32)
            for j in range(_DIM // _L):
                hv = rows_h[row, pl.ds(ch + j * _L, _L)]
                rv = rows_r[row, pl.ds(cr + j * _L, _L)]
                tv = rows_t[row, pl.ds(ct + j * _L, _L)]
                psum = psum + jnp.abs(hv + rv - tv)
            total = jnp.sum(psum) - _GAMMA
            acc = jnp.where(lane == l, total, acc)
        out_v[pl.ds(out_base + b * _L, _L)] = acc
        return carry

    lax.fori_loop(0, _CHUNK // _L, block_body, 0)


def _body(hidx_hbm, ridx_hbm, tidx_hbm, ent_hbm, rel_hbm, out_hbm,
          idx_h, idx_r, idx_t, gdx_h, gdx_r, gdx_t,
          rows_ha, rows_ra, rows_ta, rows_hb, rows_rb, rows_tb,
          out_v, sem_a, sem_b):
    wid = lax.axis_index("s") * _NC + lax.axis_index("c")
    base = wid * _ROWS_PER_W

    for c in range(_NCHUNK):
        src = pl.ds(base + c * _CHUNK, _CHUNK)
        pltpu.sync_copy(hidx_hbm.at[src], idx_h.at[c])
        pltpu.sync_copy(ridx_hbm.at[src], idx_r.at[c])
        pltpu.sync_copy(tidx_hbm.at[src], idx_t.at[c])

    def shift_body(i, carry):
        sl = pl.ds(i * _L, _L)
        for c in range(_NCHUNK):
            for idx, gdx in ((idx_h, gdx_h), (idx_r, gdx_r), (idx_t, gdx_t)):
                v = idx[c, sl]
                gdx[c, sl] = jnp.where(v >= _HALF, v - _HALF, v)
        return carry

    lax.fori_loop(0, _CHUNK // _L, shift_body, 0)

    bufs = ((rows_ha, rows_ra, rows_ta), (rows_hb, rows_rb, rows_tb))
    sems = (sem_a, sem_b)

    def fire(c, bset, sem):
        return (
            pltpu.async_copy(ent_hbm.at[gdx_h.at[c]], bset[0], sem),
            pltpu.async_copy(rel_hbm.at[gdx_r.at[c]], bset[1], sem),
            pltpu.async_copy(ent_hbm.at[gdx_t.at[c]], bset[2], sem),
        )

    cps = fire(0, bufs[0], sems[0])
    for c in range(_NCHUNK):
        nxt = None
        if c + 1 < _NCHUNK:
            nxt = fire(c + 1, bufs[(c + 1) % 2], sems[(c + 1) % 2])
        for cp in cps:
            cp.wait()
        _compute_chunk(c, *bufs[c % 2], idx_h, idx_r, idx_t,
                       out_v, c * _CHUNK)
        cps = nxt

    pltpu.sync_copy(out_v, out_hbm.at[pl.ds(base, _ROWS_PER_W)])


@functools.partial(
    pl.kernel,
    out_type=jax.ShapeDtypeStruct((_BATCH,), jnp.float32),
    scratch_types=[
        pltpu.VMEM((_NCHUNK, _CHUNK), jnp.int32),
        pltpu.VMEM((_NCHUNK, _CHUNK), jnp.int32),
        pltpu.VMEM((_NCHUNK, _CHUNK), jnp.int32),
        pltpu.VMEM((_NCHUNK, _CHUNK), jnp.int32),
        pltpu.VMEM((_NCHUNK, _CHUNK), jnp.int32),
        pltpu.VMEM((_NCHUNK, _CHUNK), jnp.int32),
        pltpu.VMEM((_CHUNK, _PAD_DIM), jnp.float32),
        pltpu.VMEM((_CHUNK, _PAD_DIM), jnp.float32),
        pltpu.VMEM((_CHUNK, _PAD_DIM), jnp.float32),
        pltpu.VMEM((_CHUNK, _PAD_DIM), jnp.float32),
        pltpu.VMEM((_CHUNK, _PAD_DIM), jnp.float32),
        pltpu.VMEM((_CHUNK, _PAD_DIM), jnp.float32),
        pltpu.VMEM((_ROWS_PER_W,), jnp.float32),
        pltpu.SemaphoreType.DMA,
        pltpu.SemaphoreType.DMA,
    ],
    mesh=plsc.VectorSubcoreMesh(core_axis_name="c", subcore_axis_name="s"),
    compiler_params=pltpu.CompilerParams(
        needs_layout_passes=False, use_tc_tiling_on_sc=False
    ),
)
def _transe_sc(*args):
    _body(*args)


def kernel(pos_sample, ent_embd, rel_embd):
    ent_t = ent_embd.T
    rel_t = rel_embd.T
    ent_p, rel_p = _transpose_tables(ent_t, ent_t, rel_t, rel_t)
    h_idx = pos_sample[:, 0]
    r_idx = pos_sample[:, 1]
    t_idx = pos_sample[:, 2]
    score = _transe_sc(h_idx, r_idx, t_idx, ent_p, rel_p)
    return score[:, None]
